# Initial kernel scaffold; baseline (speedup 1.0000x reference)
import jax
import jax.numpy as jnp
from jax.experimental import pallas as pl


def kernel(x, w1c, b1, w2d, b2, wf1, bf1, wf2p, bf2p):
    raise NotImplementedError("write your pallas kernel here")


# speedup vs baseline: 1.0161x; 1.0161x over previous
import jax
import jax.numpy as jnp
from jax.experimental import pallas as pl
from jax.experimental.pallas import tpu as pltpu

H0 = W0 = 64
C0 = 3
C1 = 32
H1 = W1 = 64
H1P = W1P = 32
C2 = 64
H2P = W2P = 16
K1 = 9 * C0
K2 = 3 * C1
NFEAT = C2 * H2P * W2P
NHID = 128
NOUT = 11
NOUT_PAD = 128

HP2, WPAD2, PADL = H1P + 2, 48, 8

_VMEM_LIMIT = 64 * 1024 * 1024


def _conv_tower_kernel(cols_ref, w1c_ref, b1_ref, w2d_ref, b2_ref, o_ref,
                       xpad2_ref):
    bt = cols_ref.shape[0]

    xpad2_ref[:, 0:1, :, :] = jnp.zeros((bt, 1, WPAD2, C1), jnp.bfloat16)
    xpad2_ref[:, HP2 - 1:HP2, :, :] = jnp.zeros((bt, 1, WPAD2, C1),
                                                jnp.bfloat16)
    xpad2_ref[:, :, 0:PADL, :] = jnp.zeros((bt, HP2, PADL, C1), jnp.bfloat16)
    xpad2_ref[:, :, PADL + W1P:WPAD2, :] = jnp.zeros(
        (bt, HP2, WPAD2 - PADL - W1P, C1), jnp.bfloat16)

    cols = cols_ref[...].reshape(bt * H1 * W1, K1)
    y1 = jnp.dot(cols, w1c_ref[...], preferred_element_type=jnp.float32)
    y1 = jnp.maximum(y1 + b1_ref[...], 0.0)

    u = y1.reshape(bt, H1, W1 // 2, 2, C1)
    u = jnp.max(u, axis=3)
    u = u.reshape(bt, H1 // 2, 2, W1 // 2, C1)
    p1 = jnp.max(u, axis=2)

    xpad2_ref[:, 1:1 + H1P, PADL:PADL + W1P, :] = p1.astype(jnp.bfloat16)
    xp2 = xpad2_ref[...]
    acc = None
    for dy in range(3):
        rows = xp2[:, dy:dy + H1P]
        win = jnp.concatenate(
            [rows[:, :, PADL - 1 + dx:PADL - 1 + dx + W1P, :]
             for dx in range(3)], axis=-1)
        d = jnp.dot(win.reshape(bt * H1P * W1P, K2), w2d_ref[dy],
                    preferred_element_type=jnp.float32)
        acc = d if acc is None else acc + d
    y2 = jnp.maximum(acc + b2_ref[...], 0.0)

    u = y2.reshape(bt, H1P, W1P // 2, 2, C2)
    u = jnp.max(u, axis=3)
    u = u.reshape(bt, H1P // 2, 2, W1P // 2, C2)
    p2 = jnp.max(u, axis=2)

    p2 = p2.reshape(bt, H2P, W2P // 2, 2, C2)
    dense = jnp.concatenate([p2[:, :, :, 0, :], p2[:, :, :, 1, :]], axis=-1)
    o_ref[...] = dense.reshape(bt, H2P * (W2P // 2),
                               2 * C2).astype(jnp.bfloat16)


def _conv_tower(cols, w1c, b1, w2d, b2, bt):
    B = cols.shape[0]
    return pl.pallas_call(
        _conv_tower_kernel,
        out_shape=jax.ShapeDtypeStruct((B, H2P * (W2P // 2), 2 * C2),
                                       jnp.bfloat16),
        grid=(B // bt,),
        in_specs=[
            pl.BlockSpec((bt, H1 * W1, K1), lambda i: (i, 0, 0)),
            pl.BlockSpec((K1, C1), lambda i: (0, 0)),
            pl.BlockSpec((1, C1), lambda i: (0, 0)),
            pl.BlockSpec((3, K2, C2), lambda i: (0, 0, 0)),
            pl.BlockSpec((1, C2), lambda i: (0, 0)),
        ],
        out_specs=pl.BlockSpec((bt, H2P * (W2P // 2), 2 * C2),
                               lambda i: (i, 0, 0)),
        scratch_shapes=[pltpu.VMEM((bt, HP2, WPAD2, C1), jnp.bfloat16)],
        compiler_params=pltpu.CompilerParams(
            dimension_semantics=("parallel",),
            vmem_limit_bytes=_VMEM_LIMIT),
    )(cols, w1c, b1, w2d, b2)


def _mlp_kernel(x_ref, w1_ref, b1_ref, w2_ref, b2_ref, o_ref):
    h = jnp.dot(x_ref[...], w1_ref[...],
                preferred_element_type=jnp.float32) + b1_ref[...]
    h = jnp.maximum(h, 0.0).astype(jnp.bfloat16)
    o_ref[...] = (jnp.dot(h, w2_ref[...],
                          preferred_element_type=jnp.float32) + b2_ref[...])


def _mlp(x, w1, b1, w2p, b2p, bm):
    B, K = x.shape
    return pl.pallas_call(
        _mlp_kernel,
        out_shape=jax.ShapeDtypeStruct((B, NOUT_PAD), jnp.float32),
        grid=(B // bm,),
        in_specs=[
            pl.BlockSpec((bm, K), lambda i: (i, 0)),
            pl.BlockSpec((K, NHID), lambda i: (0, 0)),
            pl.BlockSpec((1, NHID), lambda i: (0, 0)),
            pl.BlockSpec((NHID, NOUT_PAD), lambda i: (0, 0)),
            pl.BlockSpec((1, NOUT_PAD), lambda i: (0, 0)),
        ],
        out_specs=pl.BlockSpec((bm, NOUT_PAD), lambda i: (i, 0)),
        compiler_params=pltpu.CompilerParams(
            dimension_semantics=("parallel",),
            vmem_limit_bytes=_VMEM_LIMIT),
    )(x, w1, b1, w2p, b2p)


@jax.jit
def _forward(x_nchw, w1c, b1, w2d, b2, wf1, bf1, wf2p, bf2p):
    B = x_nchw.shape[0]
    x = jnp.transpose(x_nchw, (0, 2, 3, 1))
    xp = jnp.pad(x, ((0, 0), (1, 1), (1, 1), (0, 0)))
    cols = jnp.concatenate(
        [xp[:, dy:dy + H1, dx:dx + W1, :] for dy in range(3)
         for dx in range(3)],
        axis=-1).reshape(B, H1 * W1, K1).astype(jnp.bfloat16)

    feats = _conv_tower(cols, w1c.astype(jnp.bfloat16), b1,
                        w2d.astype(jnp.bfloat16), b2, bt=4)
    feats = feats.reshape(B, NFEAT)
    logits = _mlp(feats, wf1.astype(jnp.bfloat16), bf1,
                  wf2p.astype(jnp.bfloat16), bf2p, bm=64)
    return logits[:, :NOUT]


def kernel(x, w1c, b1, w2d, b2, wf1, bf1, wf2p, bf2p):
    return _forward(x, w1c, b1, w2d, b2, wf1, bf1, wf2p, bf2p)


# speedup vs baseline: 1.7638x; 1.7359x over previous
import jax
import jax.numpy as jnp
import numpy as np
from jax.experimental import pallas as pl
from jax.experimental.pallas import tpu as pltpu

H0 = W0 = 64
C0 = 3
C1 = 32
H1P = W1P = 32
C2 = 64
H2P = W2P = 16
NFEAT = C2 * H2P * W2P
NHID = 128
NOUT = 11
NOUT_PAD = 128

KW1 = C0 * 3 * 10
KW2 = 6 * C1

_VMEM_LIMIT = 48 * 1024 * 1024

_S8_PERM = np.array([0, 2, 4, 6, 1, 3, 5, 7])
_S4_PERM = np.array([0, 2, 1, 3])


def _conv_tower_kernel(cw_ref, w1_ref, b1_ref, w2_ref, b2_ref, o_ref, p1s_ref):
    bt = cw_ref.shape[0]

    cols1 = cw_ref[...].reshape(bt * 64 * 8, KW1)
    y1 = jnp.dot(cols1, w1_ref[...], preferred_element_type=jnp.float32)
    y1 = jnp.maximum(y1 + b1_ref[...], 0.0)

    px = jnp.maximum(y1[:, :128], y1[:, 128:])
    px = px.reshape(bt, 32, 2, 8, 128)
    p1 = jnp.maximum(px[:, :, 0], px[:, :, 1])

    zc = jnp.zeros((bt, 32, 1, 32), jnp.float32)
    xm1 = jnp.concatenate([zc, p1[:, :, :7, 96:128]], axis=2)
    xp1 = jnp.concatenate([p1[:, :, 1:, 0:32], zc], axis=2)
    p1s_ref[:, 0:8] = jnp.zeros((bt, 8, KW2), jnp.bfloat16)
    p1s_ref[:, 264:272] = jnp.zeros((bt, 8, KW2), jnp.bfloat16)
    p1s_ref[:, 8:264, 0:128] = p1.reshape(bt, 256, 128).astype(jnp.bfloat16)
    p1s_ref[:, 8:264, 128:160] = xm1.reshape(bt, 256, 32).astype(jnp.bfloat16)
    p1s_ref[:, 8:264, 160:192] = xp1.reshape(bt, 256, 32).astype(jnp.bfloat16)

    acc = None
    for dy in range(3):
        cols2 = p1s_ref[:, dy * 8:dy * 8 + 256, :]
        d = jnp.dot(cols2.reshape(bt * 256, KW2), w2_ref[dy],
                    preferred_element_type=jnp.float32)
        acc = d if acc is None else acc + d
    y2 = jnp.maximum(acc + b2_ref[...], 0.0)

    qx = jnp.maximum(y2[:, :128], y2[:, 128:])
    qx = qx.reshape(bt, 16, 2, 8, 128)
    p2 = jnp.maximum(qx[:, :, 0], qx[:, :, 1])

    o_ref[...] = p2.reshape(bt, 128, 128).astype(jnp.bfloat16)


def _conv_tower(cw, w1w, b1w, w2w, b2w, bt):
    B = cw.shape[0]
    return pl.pallas_call(
        _conv_tower_kernel,
        out_shape=jax.ShapeDtypeStruct((B, 128, 128), jnp.bfloat16),
        grid=(B // bt,),
        in_specs=[
            pl.BlockSpec((bt, 64, 8, KW1), lambda i: (i, 0, 0, 0)),
            pl.BlockSpec((KW1, 256), lambda i: (0, 0)),
            pl.BlockSpec((1, 256), lambda i: (0, 0)),
            pl.BlockSpec((3, KW2, 256), lambda i: (0, 0, 0)),
            pl.BlockSpec((1, 256), lambda i: (0, 0)),
        ],
        out_specs=pl.BlockSpec((bt, 128, 128), lambda i: (i, 0, 0)),
        scratch_shapes=[pltpu.VMEM((bt, 272, KW2), jnp.bfloat16)],
        compiler_params=pltpu.CompilerParams(
            dimension_semantics=("parallel",),
            vmem_limit_bytes=_VMEM_LIMIT),
    )(cw, w1w, b1w, w2w, b2w)


def _mlp_kernel(x_ref, w1_ref, b1_ref, w2_ref, b2_ref, o_ref):
    h = jnp.dot(x_ref[...], w1_ref[...],
                preferred_element_type=jnp.float32) + b1_ref[...]
    h = jnp.maximum(h, 0.0).astype(jnp.bfloat16)
    o_ref[...] = (jnp.dot(h, w2_ref[...],
                          preferred_element_type=jnp.float32) + b2_ref[...])


def _mlp(x, w1, b1, w2p, b2p, bm):
    B, K = x.shape
    return pl.pallas_call(
        _mlp_kernel,
        out_shape=jax.ShapeDtypeStruct((B, NOUT_PAD), jnp.float32),
        grid=(B // bm,),
        in_specs=[
            pl.BlockSpec((bm, K), lambda i: (i, 0)),
            pl.BlockSpec((K, NHID), lambda i: (0, 0)),
            pl.BlockSpec((1, NHID), lambda i: (0, 0)),
            pl.BlockSpec((NHID, NOUT_PAD), lambda i: (0, 0)),
            pl.BlockSpec((1, NOUT_PAD), lambda i: (0, 0)),
        ],
        out_specs=pl.BlockSpec((bm, NOUT_PAD), lambda i: (i, 0)),
        compiler_params=pltpu.CompilerParams(
            dimension_semantics=("parallel",),
            vmem_limit_bytes=_VMEM_LIMIT),
    )(x, w1, b1, w2p, b2p)


def _build_cols_wide(x):
    B = x.shape[0]
    xpn = jnp.pad(x, ((0, 0), (0, 0), (1, 1), (1, 1)))
    slabs = [xpn[:, :, dy:dy + 64, dxu:dxu + 57:8]
             for dy in range(3) for dxu in range(10)]
    cw = jnp.stack(slabs, axis=-1)
    return cw.transpose(0, 2, 3, 1, 4).reshape(B, 64, 8, KW1).astype(
        jnp.bfloat16)


def _build_w1w(w1c):
    w1r = w1c.reshape(3, 3, C0, C1)
    m = np.zeros((3, 10, 8), np.float32)
    for s in range(8):
        for dx in range(3):
            m[dx, s + dx, s] = 1.0
    t = jnp.einsum('YDIC,DUS->IYUSC', w1r, jnp.asarray(m))
    t = t[:, :, :, _S8_PERM, :]
    return t.reshape(KW1, 8 * C1).astype(jnp.bfloat16)


def _build_w2w(w2d):
    w2r = w2d.reshape(3, 3, C1, C2)
    m = np.zeros((3, 6, 4), np.float32)
    for s in range(4):
        for dx in range(3):
            m[dx, s + dx, s] = 1.0
    t = jnp.einsum('YDIC,DUS->YUISC', w2r, jnp.asarray(m))
    t = jnp.concatenate([t[:, 1:5], t[:, 0:1], t[:, 5:6]], axis=1)
    t = t[:, :, :, _S4_PERM, :]
    return t.reshape(3, KW2, 4 * C2).astype(jnp.bfloat16)


@jax.jit
def _forward(x, w1c, b1, w2d, b2, wf1, bf1, wf2p, bf2p):
    B = x.shape[0]
    cw = _build_cols_wide(x)
    w1w = _build_w1w(w1c)
    w2w = _build_w2w(w2d)
    b1w = jnp.tile(b1, (1, 8))
    b2w = jnp.tile(b2, (1, 4))

    feats = _conv_tower(cw, w1w, b1w, w2w, b2w, bt=min(8, B))
    feats = feats.reshape(B, NFEAT)
    logits = _mlp(feats, wf1.astype(jnp.bfloat16), bf1,
                  wf2p.astype(jnp.bfloat16), bf2p, bm=min(64, B))
    return logits[:, :NOUT]


def kernel(x, w1c, b1, w2d, b2, wf1, bf1, wf2p, bf2p):
    return _forward(x, w1c, b1, w2d, b2, wf1, bf1, wf2p, bf2p)


# speedup vs baseline: 3.6329x; 2.0597x over previous
import jax
import jax.numpy as jnp
import numpy as np
from jax.experimental import pallas as pl
from jax.experimental.pallas import tpu as pltpu

H0 = W0 = 64
C0 = 3
C1 = 32
H1P = W1P = 32
C2 = 64
H2P = W2P = 16
NFEAT = C2 * H2P * W2P
NHID = 128
NOUT = 11
NOUT_PAD = 128

KW1S = 30
KW2 = 6 * C1

_VMEM_LIMIT = 48 * 1024 * 1024

_S8_PERM = np.array([0, 2, 4, 6, 1, 3, 5, 7])
_S4_PERM = np.array([0, 2, 1, 3])


def _conv_tower_kernel(x_ref, w1_ref, b1_ref, w2_ref, b2_ref, o_ref, xs_ref,
                       p1s_ref):
    bt = x_ref.shape[0]

    x = x_ref[...]
    tcs = [x[:, ci].reshape(bt, 64, 8, 8) for ci in range(3)]
    xr = jnp.concatenate(tcs, axis=-1)
    zc3 = jnp.zeros((bt, 64, 1, 3), jnp.bfloat16)
    em = jnp.concatenate([t[:, :, :7, 7:8] for t in tcs], axis=-1)
    ep = jnp.concatenate([t[:, :, 1:, 0:1] for t in tcs], axis=-1)
    xme = jnp.concatenate([zc3, em], axis=2)
    xpe = jnp.concatenate([ep, zc3], axis=2)
    xs_ref[:, 0:8] = jnp.zeros((bt, 8, KW1S), jnp.bfloat16)
    xs_ref[:, 520:528] = jnp.zeros((bt, 8, KW1S), jnp.bfloat16)
    xs_ref[:, 8:520, 0:24] = xr.reshape(bt, 512, 24)
    xs_ref[:, 8:520, 24:27] = xme.reshape(bt, 512, 3)
    xs_ref[:, 8:520, 27:30] = xpe.reshape(bt, 512, 3)

    y1 = None
    for dy in range(3):
        cols1 = xs_ref[:, dy * 8:dy * 8 + 512, :]
        d = jnp.dot(cols1.reshape(bt * 512, KW1S), w1_ref[dy],
                    preferred_element_type=jnp.float32)
        y1 = d if y1 is None else y1 + d
    y1 = jnp.maximum(y1 + b1_ref[...], 0.0)

    px = jnp.maximum(y1[:, :128], y1[:, 128:])
    px = px.reshape(bt, 32, 2, 8, 128)
    p1 = jnp.maximum(px[:, :, 0], px[:, :, 1])

    zc = jnp.zeros((bt, 32, 1, 32), jnp.float32)
    xm1 = jnp.concatenate([zc, p1[:, :, :7, 96:128]], axis=2)
    xp1 = jnp.concatenate([p1[:, :, 1:, 0:32], zc], axis=2)
    p1s_ref[:, 0:8] = jnp.zeros((bt, 8, KW2), jnp.bfloat16)
    p1s_ref[:, 264:272] = jnp.zeros((bt, 8, KW2), jnp.bfloat16)
    p1s_ref[:, 8:264, 0:128] = p1.reshape(bt, 256, 128).astype(jnp.bfloat16)
    p1s_ref[:, 8:264, 128:160] = xm1.reshape(bt, 256, 32).astype(jnp.bfloat16)
    p1s_ref[:, 8:264, 160:192] = xp1.reshape(bt, 256, 32).astype(jnp.bfloat16)

    acc = None
    for dy in range(3):
        cols2 = p1s_ref[:, dy * 8:dy * 8 + 256, :]
        d = jnp.dot(cols2.reshape(bt * 256, KW2), w2_ref[dy],
                    preferred_element_type=jnp.float32)
        acc = d if acc is None else acc + d
    y2 = jnp.maximum(acc + b2_ref[...], 0.0)

    qx = jnp.maximum(y2[:, :128], y2[:, 128:])
    qx = qx.reshape(bt, 16, 2, 8, 128)
    p2 = jnp.maximum(qx[:, :, 0], qx[:, :, 1])

    o_ref[...] = p2.reshape(bt, 128, 128).astype(jnp.bfloat16)


def _conv_tower(x, w1s, b1w, w2w, b2w, bt):
    B = x.shape[0]
    return pl.pallas_call(
        _conv_tower_kernel,
        out_shape=jax.ShapeDtypeStruct((B, 128, 128), jnp.bfloat16),
        grid=(B // bt,),
        in_specs=[
            pl.BlockSpec((bt, 3, 64, 64), lambda i: (i, 0, 0, 0)),
            pl.BlockSpec((3, KW1S, 256), lambda i: (0, 0, 0)),
            pl.BlockSpec((1, 256), lambda i: (0, 0)),
            pl.BlockSpec((3, KW2, 256), lambda i: (0, 0, 0)),
            pl.BlockSpec((1, 256), lambda i: (0, 0)),
        ],
        out_specs=pl.BlockSpec((bt, 128, 128), lambda i: (i, 0, 0)),
        scratch_shapes=[pltpu.VMEM((bt, 528, KW1S), jnp.bfloat16),
                        pltpu.VMEM((bt, 272, KW2), jnp.bfloat16)],
        compiler_params=pltpu.CompilerParams(
            dimension_semantics=("parallel",),
            vmem_limit_bytes=_VMEM_LIMIT),
    )(x, w1s, b1w, w2w, b2w)


def _mlp_kernel(x_ref, w1_ref, b1_ref, w2_ref, b2_ref, o_ref):
    h = jnp.dot(x_ref[...], w1_ref[...],
                preferred_element_type=jnp.float32) + b1_ref[...]
    h = jnp.maximum(h, 0.0).astype(jnp.bfloat16)
    o_ref[...] = (jnp.dot(h, w2_ref[...],
                          preferred_element_type=jnp.float32) + b2_ref[...])


def _mlp(x, w1, b1, w2p, b2p, bm):
    B, K = x.shape
    return pl.pallas_call(
        _mlp_kernel,
        out_shape=jax.ShapeDtypeStruct((B, NOUT_PAD), jnp.float32),
        grid=(B // bm,),
        in_specs=[
            pl.BlockSpec((bm, K), lambda i: (i, 0)),
            pl.BlockSpec((K, NHID), lambda i: (0, 0)),
            pl.BlockSpec((1, NHID), lambda i: (0, 0)),
            pl.BlockSpec((NHID, NOUT_PAD), lambda i: (0, 0)),
            pl.BlockSpec((1, NOUT_PAD), lambda i: (0, 0)),
        ],
        out_specs=pl.BlockSpec((bm, NOUT_PAD), lambda i: (i, 0)),
        compiler_params=pltpu.CompilerParams(
            dimension_semantics=("parallel",),
            vmem_limit_bytes=_VMEM_LIMIT),
    )(x, w1, b1, w2p, b2p)


def _build_w1s(w1c):
    w1r = w1c.reshape(3, 3, C0, C1)
    m = np.zeros((3, 10, 8), np.float32)
    for s in range(8):
        for dx in range(3):
            m[dx, s + dx, s] = 1.0
    t = jnp.einsum('YDIC,DJS->YIJSC', w1r, jnp.asarray(m))
    t = t[:, :, :, _S8_PERM, :]
    main = t[:, :, 1:9].reshape(3, 24, 8 * C1)
    em = t[:, :, 0].reshape(3, 3, 8 * C1)
    ep = t[:, :, 9].reshape(3, 3, 8 * C1)
    return jnp.concatenate([main, em, ep], axis=1).astype(jnp.bfloat16)


def _build_w2w(w2d):
    w2r = w2d.reshape(3, 3, C1, C2)
    m = np.zeros((3, 6, 4), np.float32)
    for s in range(4):
        for dx in range(3):
            m[dx, s + dx, s] = 1.0
    t = jnp.einsum('YDIC,DUS->YUISC', w2r, jnp.asarray(m))
    t = jnp.concatenate([t[:, 1:5], t[:, 0:1], t[:, 5:6]], axis=1)
    t = t[:, :, :, _S4_PERM, :]
    return t.reshape(3, KW2, 4 * C2).astype(jnp.bfloat16)


@jax.jit
def _forward(x, w1c, b1, w2d, b2, wf1, bf1, wf2p, bf2p):
    B = x.shape[0]
    w1s = _build_w1s(w1c)
    w2w = _build_w2w(w2d)
    b1w = jnp.tile(b1, (1, 8))
    b2w = jnp.tile(b2, (1, 4))

    feats = _conv_tower(x.astype(jnp.bfloat16), w1s, b1w, w2w, b2w,
                        bt=min(8, B))
    feats = feats.reshape(B, NFEAT)
    logits = _mlp(feats, wf1.astype(jnp.bfloat16), bf1,
                  wf2p.astype(jnp.bfloat16), bf2p, bm=min(64, B))
    return logits[:, :NOUT]


def kernel(x, w1c, b1, w2d, b2, wf1, bf1, wf2p, bf2p):
    return _forward(x, w1c, b1, w2d, b2, wf1, bf1, wf2p, bf2p)


# speedup vs baseline: 4.7290x; 1.3017x over previous
import jax
import jax.numpy as jnp
import numpy as np
from jax.experimental import pallas as pl
from jax.experimental.pallas import tpu as pltpu

C0, C1, C2 = 3, 32, 64
NFEAT = 16384
NHID = 128
NOUT = 11
NOUT_PAD = 128

K1B = 3 * C0 * 64 + 1
K2B = 32 * C1
N1 = 64 * C1
N2 = 32 * C2

_VMEM_LIMIT = 60 * 1024 * 1024


def _shift_dn(a, h):
    lo, hi = a[:, :, 0:h], a[:, :, h:2 * h]
    z = jnp.zeros_like(a[:, :, 0:1])
    return jnp.concatenate(
        [jnp.concatenate([z, hi[:, :, :h - 1]], axis=2), lo], axis=2)


def _shift_up(a, h):
    lo, hi = a[:, :, 0:h], a[:, :, h:2 * h]
    z = jnp.zeros_like(a[:, :, 0:1])
    return jnp.concatenate(
        [hi, jnp.concatenate([lo[:, :, 1:], z], axis=2)], axis=2)


def _conv_tower_kernel(x_ref, w1_ref, w2_ref, b2_ref, o_ref):
    bt = x_ref.shape[0]
    xx = x_ref[...]

    xh0, xh1 = xx[:, :, 0:32], xx[:, :, 32:64]
    ym1 = jnp.concatenate([_shift_dn(xh1, 16), xh0], axis=2)
    yp1 = jnp.concatenate([xh1, _shift_up(xh0, 16)], axis=2)
    ones = jnp.ones((bt, 64, 1), jnp.bfloat16)
    lhs1 = jnp.concatenate(
        [ym1[:, 0], ym1[:, 1], ym1[:, 2],
         xx[:, 0], xx[:, 1], xx[:, 2],
         yp1[:, 0], yp1[:, 1], yp1[:, 2], ones], axis=-1)

    y1 = jnp.dot(lhs1.reshape(bt * 64, K1B), w1_ref[...],
                 preferred_element_type=jnp.float32)

    px = jnp.maximum(y1[:, :1024], y1[:, 1024:])
    px = px.reshape(bt, 2, 32, 1024)
    p1 = jnp.maximum(px[:, 0], px[:, 1])
    p1 = jnp.maximum(p1, 0.0).astype(jnp.bfloat16)

    p1 = p1.reshape(bt, 1, 32, K2B)
    lhs2 = jnp.concatenate(
        [_shift_dn(p1, 16)[:, 0], p1[:, 0], _shift_up(p1, 16)[:, 0]],
        axis=-1)
    y2 = jnp.dot(lhs2.reshape(bt * 32, 3 * K2B), w2_ref[...],
                 preferred_element_type=jnp.float32)

    qx = jnp.maximum(y2[:, :1024], y2[:, 1024:])
    qx = qx.reshape(bt, 2, 16, 1024)
    p2 = jnp.maximum(qx[:, 0], qx[:, 1])
    p2 = jnp.maximum(p2 + b2_ref[...], 0.0)

    o_ref[...] = p2.astype(jnp.bfloat16)


def _conv_tower(xpp, w1b, w2b, b2p, bt):
    B = xpp.shape[0]
    return pl.pallas_call(
        _conv_tower_kernel,
        out_shape=jax.ShapeDtypeStruct((B, 16, 1024), jnp.bfloat16),
        grid=(B // bt,),
        in_specs=[
            pl.BlockSpec((bt, 3, 64, 64), lambda i: (i, 0, 0, 0)),
            pl.BlockSpec((K1B, N1), lambda i: (0, 0)),
            pl.BlockSpec((3 * K2B, N2), lambda i: (0, 0)),
            pl.BlockSpec((1, 1024), lambda i: (0, 0)),
        ],
        out_specs=pl.BlockSpec((bt, 16, 1024), lambda i: (i, 0, 0)),
        compiler_params=pltpu.CompilerParams(
            dimension_semantics=("parallel",),
            vmem_limit_bytes=_VMEM_LIMIT),
    )(xpp, w1b, w2b, b2p)


def _mlp_kernel(x_ref, w1_ref, b1_ref, w2_ref, b2_ref, o_ref):
    h = jnp.dot(x_ref[...], w1_ref[...],
                preferred_element_type=jnp.float32) + b1_ref[...]
    h = jnp.maximum(h, 0.0).astype(jnp.bfloat16)
    o_ref[...] = (jnp.dot(h, w2_ref[...],
                          preferred_element_type=jnp.float32) + b2_ref[...])


def _mlp(x, w1, b1, w2p, b2p, bm):
    B, K = x.shape
    return pl.pallas_call(
        _mlp_kernel,
        out_shape=jax.ShapeDtypeStruct((B, NOUT_PAD), jnp.float32),
        grid=(B // bm,),
        in_specs=[
            pl.BlockSpec((bm, K), lambda i: (i, 0)),
            pl.BlockSpec((K, NHID), lambda i: (0, 0)),
            pl.BlockSpec((1, NHID), lambda i: (0, 0)),
            pl.BlockSpec((NHID, NOUT_PAD), lambda i: (0, 0)),
            pl.BlockSpec((1, NOUT_PAD), lambda i: (0, 0)),
        ],
        out_specs=pl.BlockSpec((bm, NOUT_PAD), lambda i: (i, 0)),
        compiler_params=pltpu.CompilerParams(
            dimension_semantics=("parallel",),
            vmem_limit_bytes=_VMEM_LIMIT),
    )(x, w1, b1, w2p, b2p)


def _band(nx, dtype=np.float32):
    b = np.zeros((3, nx, nx), dtype)
    for dx in range(3):
        for xo in range(nx):
            xi = xo + dx - 1
            if 0 <= xi < nx:
                b[dx, xi, xo] = 1.0
    return b


def _parity(nx):
    return np.concatenate([np.arange(0, nx, 2), np.arange(1, nx, 2)])


def _build_w1b(w1c, b1):
    w1r = w1c.reshape(3, 3, C0, C1)
    t = jnp.einsum('YDIC,DXO->YIXOC', w1r, jnp.asarray(_band(64)))
    t = t[:, :, :, _parity(64), :]
    main = t.reshape(9 * 64, N1)
    return jnp.concatenate([main, jnp.tile(b1, (1, 64))],
                           axis=0).astype(jnp.bfloat16)


def _build_w2b(w2d):
    w2r = w2d.reshape(3, 3, C1, C2)
    t = jnp.einsum('YDIC,DXO->YXIOC', w2r, jnp.asarray(_band(32)))
    t = t[:, :, :, _parity(32), :]
    return t.reshape(3 * K2B, N2).astype(jnp.bfloat16)


@jax.jit
def _forward(x, w1c, b1, w2d, b2, wf1, bf1, wf2p, bf2p):
    B = x.shape[0]
    xb = x.astype(jnp.bfloat16)
    perm = np.array([4 * y4 + 2 * p2 + p
                     for p in (0, 1) for p2 in (0, 1) for y4 in range(16)])
    xpp = xb[:, :, perm]

    w1b = _build_w1b(w1c, b1)
    w2b = _build_w2b(w2d)
    b2p = jnp.tile(b2, (1, 16))

    feats = _conv_tower(xpp, w1b, w2b, b2p, bt=min(8, B))
    feats = feats.reshape(B, NFEAT)
    logits = _mlp(feats, wf1.astype(jnp.bfloat16), bf1,
                  wf2p.astype(jnp.bfloat16), bf2p, bm=min(64, B))
    return logits[:, :NOUT]


def kernel(x, w1c, b1, w2d, b2, wf1, bf1, wf2p, bf2p):
    return _forward(x, w1c, b1, w2d, b2, wf1, bf1, wf2p, bf2p)
